# baseline (speedup 1.0000x reference)
import functools

import jax
import jax.numpy as jnp
from jax import lax
from jax.experimental import pallas as pl
from jax.experimental.pallas import tpu as pltpu
from jax.experimental.pallas import tpu_sc as plsc

V = 10000
R = 10016
D = 128
DH = 64
E = 320000
RPT = R // 16

CH = 80
NBUF = 2
NCHT = 250
CHT = 16
NGRP = NCHT // NBUF

CHD = 128
NCHD = 80
EPADD = 32 * NCHD * CHD


def _deg_kernel(gidx3, sidx3, ones_rows, zeros16):
  mesh = plsc.VectorSubcoreMesh(core_axis_name="c", subcore_axis_name="s")

  @functools.partial(
      pl.kernel,
      mesh=mesh,
      out_type=jax.ShapeDtypeStruct((2, 2, R, 16), jnp.float32),
      compiler_params=pltpu.CompilerParams(use_tc_tiling_on_sc=False),
      scratch_types=[
          pltpu.VMEM((NCHD, CHD), jnp.int32),
          pltpu.VMEM((NCHD, CHD), jnp.int32),
          pltpu.VMEM((CHD, 16), jnp.float32),
          pltpu.VMEM_SHARED((R, 16), jnp.float32),
          pltpu.VMEM_SHARED((R, 16), jnp.float32),
      ] + [pltpu.SemaphoreType.DMA] * 4,
  )
  def k(gidx_hbm, sidx_hbm, ones_hbm, zeros_hbm, out_hbm,
        gi2, si2, ones_v, acce, accv, e0, e1, v0, v1):
    seme = [e0, e1]
    semv = [v0, v1]
    c = lax.axis_index("c")
    s = lax.axis_index("s")
    wid = s * 2 + c
    rslc = pl.ds(s * RPT, RPT)
    pltpu.sync_copy(zeros_hbm.at[rslc], acce.at[rslc])
    pltpu.sync_copy(zeros_hbm.at[rslc], accv.at[rslc])
    pltpu.sync_copy(ones_hbm, ones_v)
    pltpu.sync_copy(gidx_hbm.at[wid], gi2)
    pltpu.sync_copy(sidx_hbm.at[wid], si2)
    plsc.subcore_barrier()

    def estart(t, b):
      pltpu.async_copy(ones_v, acce.at[si2.at[t]], seme[b], add=True)

    def vstart(t, b):
      pltpu.async_copy(ones_v, accv.at[gi2.at[t]], semv[b], add=True)

    def ewait(t, b):
      pltpu.make_async_copy(ones_v, acce.at[si2.at[t]], seme[b]).wait()

    def vwait(t, b):
      pltpu.make_async_copy(ones_v, accv.at[gi2.at[t]], semv[b]).wait()

    for b in range(2):
      estart(b, b)
      vstart(b, b)

    def group(m, carry):
      t0 = 2 * m
      for b in range(2):
        ewait(t0 - 2 + b, b)
        vwait(t0 - 2 + b, b)
        estart(t0 + b, b)
        vstart(t0 + b, b)
      return carry

    lax.fori_loop(1, NCHD // 2, group, 0)
    for b in range(2):
      ewait(NCHD - 2 + b, b)
      vwait(NCHD - 2 + b, b)
    plsc.subcore_barrier()
    pltpu.sync_copy(acce.at[rslc], out_hbm.at[0, c, rslc])
    pltpu.sync_copy(accv.at[rslc], out_hbm.at[1, c, rslc])

  return k(gidx3, sidx3, ones_rows, zeros16)


def _sc_both_passes(table, gidx3, sidx3, zeros_half, dege):
  mesh = plsc.VectorSubcoreMesh(core_axis_name="c", subcore_axis_name="s")

  @functools.partial(
      pl.kernel,
      mesh=mesh,
      out_type=jax.ShapeDtypeStruct((2, R, DH), jnp.float32),
      compiler_params=pltpu.CompilerParams(use_tc_tiling_on_sc=False),
      scratch_types=[
          pltpu.VMEM((NCHT, CH), jnp.int32),
          pltpu.VMEM((NCHT, CH), jnp.int32),
      ] + [pltpu.VMEM((CH, DH), jnp.float32)] * NBUF
        + [pltpu.VMEM((CHT, 16), jnp.float32),
           pltpu.VMEM((CHT, 16), jnp.float32),
           pltpu.VMEM_SHARED((R, DH), jnp.float32),
           pltpu.VMEM_SHARED((R, DH), jnp.float32)]
        + [pltpu.SemaphoreType.DMA] * (2 * NBUF),
  )
  def k(table_hbm, gidx_hbm, sidx_hbm, zeros_hbm, dege_hbm, out_hbm,
        *scratch):
    gi2, si2 = scratch[0], scratch[1]
    rows = list(scratch[2:2 + NBUF])
    d0buf = scratch[2 + NBUF]
    d1buf = scratch[3 + NBUF]
    tbl = scratch[4 + NBUF]
    acc = scratch[5 + NBUF]
    gsem = list(scratch[6 + NBUF:6 + 2 * NBUF])
    ssem = list(scratch[6 + 2 * NBUF:6 + 3 * NBUF])
    c = lax.axis_index("c")
    s = lax.axis_index("s")
    rslc = pl.ds(s * RPT, RPT)
    pltpu.sync_copy(table_hbm.at[rslc, pl.ds(c * DH, DH)], tbl.at[rslc])
    pltpu.sync_copy(zeros_hbm.at[rslc], acc.at[rslc])
    pltpu.sync_copy(gidx_hbm.at[s], gi2)
    pltpu.sync_copy(sidx_hbm.at[s], si2)
    plsc.subcore_barrier()

    def run_pass(gidx, sidx):
      def gstart(t, b):
        return pltpu.async_copy(tbl.at[gidx.at[t]], rows[b], gsem[b])

      def sstart(t, b):
        return pltpu.async_copy(rows[b], acc.at[sidx.at[t]], ssem[b],
                                add=True)

      def swait(t, b):
        pltpu.make_async_copy(rows[b], acc.at[sidx.at[t]], ssem[b]).wait()

      gd = [gstart(b, b) for b in range(NBUF)]
      for b in range(NBUF):
        gd[b].wait()
        sstart(b, b)

      def group(g, carry):
        t0 = g * NBUF
        gd = []
        for b in range(NBUF):
          swait(t0 - NBUF + b, b)
          gd.append(gstart(t0 + b, b))
        for b in range(NBUF):
          gd[b].wait()
          sstart(t0 + b, b)
        return carry

      lax.fori_loop(1, NGRP, group, 0)
      for b in range(NBUF):
        swait((NGRP - 1) * NBUF + b, b)

    run_pass(gi2, si2)
    plsc.subcore_barrier()

    def transform(row0, n):
      pltpu.sync_copy(acc.at[pl.ds(row0, n)], rows[0].at[pl.ds(0, n)])
      pltpu.sync_copy(dege_hbm.at[0, pl.ds(row0, n)], d0buf.at[pl.ds(0, n)])
      pltpu.sync_copy(dege_hbm.at[1, pl.ds(row0, n)], d1buf.at[pl.ds(0, n)])

      def row_body(r, carry):
        dvec = d0buf[r, pl.ds(0, 16)] + d1buf[r, pl.ds(0, 16)]
        invvec = jnp.where(dvec > 0.0, 1.0 / dvec, 0.0)
        inv = invvec[0]
        for kcol in range(DH // 16):
          cs = pl.ds(kcol * 16, 16)
          rows[1][r, cs] = jnp.maximum(rows[0][r, cs] * inv, 0.0)
        return carry

      lax.fori_loop(0, n, row_body, 0)
      pltpu.sync_copy(rows[1].at[pl.ds(0, n)], tbl.at[pl.ds(row0, n)])

    base = s * RPT
    nfull = RPT // CHT
    for ci in range(nfull):
      transform(base + ci * CHT, CHT)
    tail = RPT - nfull * CHT
    if tail:
      transform(base + nfull * CHT, tail)
    pltpu.sync_copy(zeros_hbm.at[rslc], acc.at[rslc])
    plsc.subcore_barrier()

    run_pass(si2, gi2)
    plsc.subcore_barrier()
    pltpu.sync_copy(acc.at[rslc], out_hbm.at[c, rslc])

  return k(table, gidx3, sidx3, zeros_half, dege)


def _combine2(q0, q1, d0, d1):
  def body(q0_ref, q1_ref, d0_ref, d1_ref, o_ref):
    deg = d0_ref[...][:, :1] + d1_ref[...][:, :1]
    inv = jnp.where(deg > 0.0, 1.0 / deg, 0.0)
    y0 = jnp.maximum(q0_ref[...] * inv, 0.0)
    y1 = jnp.maximum(q1_ref[...] * inv, 0.0)
    n2 = jnp.sum(y0 * y0, axis=1, keepdims=True) + jnp.sum(
        y1 * y1, axis=1, keepdims=True)
    scale = 1.0 / jnp.maximum(jnp.sqrt(n2), 1e-12)
    o_ref[:, :DH] = y0 * scale
    o_ref[:, DH:] = y1 * scale

  grid = 25
  blk = V // grid
  return pl.pallas_call(
      body,
      grid=(grid,),
      in_specs=[pl.BlockSpec((blk, DH), lambda i: (i, 0)),
                pl.BlockSpec((blk, DH), lambda i: (i, 0)),
                pl.BlockSpec((blk, 16), lambda i: (i, 0)),
                pl.BlockSpec((blk, 16), lambda i: (i, 0))],
      out_specs=pl.BlockSpec((blk, D), lambda i: (i, 0)),
      out_shape=jax.ShapeDtypeStruct((V, D), jnp.float32),
  )(q0, q1, d0, d1)


def _tile_layout(idx):
  cap = 16 * NCHT * CH
  flat = jnp.concatenate([idx, jnp.full((cap - E,), V, jnp.int32)])
  return flat.reshape(16, NCHT, CH)


def kernel(x, edge):
  edge_j = edge[0]
  edge_i = edge[1]

  gj = _tile_layout(edge_j)
  si = _tile_layout(edge_i)

  npadd = EPADD - E
  shp = (32, NCHD, CHD)
  gd = jnp.concatenate([edge_j, jnp.full((npadd,), V, jnp.int32)]).reshape(shp)
  sd = jnp.concatenate([edge_i, jnp.full((npadd,), V, jnp.int32)]).reshape(shp)

  zeros_half = jnp.zeros((R, DH), jnp.float32)
  zeros16 = jnp.zeros((R, 16), jnp.float32)
  ones_rows = jnp.zeros((CHD, 16), jnp.float32).at[:, 0].set(1.0)
  xa = jnp.zeros((R, D), jnp.float32).at[:V].set(x)

  deg = _deg_kernel(gd, sd, ones_rows, zeros16)
  q = _sc_both_passes(xa, gj, si, zeros_half, deg[0])
  return _combine2(q[0], q[1], deg[1, 0], deg[1, 1])


# speedup vs baseline: 1.0818x; 1.0818x over previous
import functools

import jax
import jax.numpy as jnp
from jax import lax
from jax.experimental import pallas as pl
from jax.experimental.pallas import tpu as pltpu
from jax.experimental.pallas import tpu_sc as plsc

V = 10000
R = 10016
D = 128
DH = 64
E = 320000
RPT = R // 16

CH = 64
NBUF = 2
NCHT = 314
CHT = 64
NGRP = NCHT // NBUF

CHD = 128
NCHD = 80
EPADD = 32 * NCHD * CHD


def _deg_kernel(gidx3, sidx3, ones_rows, zeros16):
  mesh = plsc.VectorSubcoreMesh(core_axis_name="c", subcore_axis_name="s")

  @functools.partial(
      pl.kernel,
      mesh=mesh,
      out_type=jax.ShapeDtypeStruct((2, 2, R, 16), jnp.float32),
      compiler_params=pltpu.CompilerParams(use_tc_tiling_on_sc=False),
      scratch_types=[
          pltpu.VMEM((NCHD, CHD), jnp.int32),
          pltpu.VMEM((NCHD, CHD), jnp.int32),
          pltpu.VMEM((CHD, 16), jnp.float32),
          pltpu.VMEM_SHARED((R, 16), jnp.float32),
          pltpu.VMEM_SHARED((R, 16), jnp.float32),
      ] + [pltpu.SemaphoreType.DMA] * 4,
  )
  def k(gidx_hbm, sidx_hbm, ones_hbm, zeros_hbm, out_hbm,
        gi2, si2, ones_v, acce, accv, e0, e1, v0, v1):
    seme = [e0, e1]
    semv = [v0, v1]
    c = lax.axis_index("c")
    s = lax.axis_index("s")
    wid = s * 2 + c
    rslc = pl.ds(s * RPT, RPT)
    pltpu.sync_copy(zeros_hbm.at[rslc], acce.at[rslc])
    pltpu.sync_copy(zeros_hbm.at[rslc], accv.at[rslc])
    pltpu.sync_copy(ones_hbm, ones_v)
    pltpu.sync_copy(gidx_hbm.at[wid], gi2)
    pltpu.sync_copy(sidx_hbm.at[wid], si2)
    plsc.subcore_barrier()

    def estart(t, b):
      pltpu.async_copy(ones_v, acce.at[si2.at[t]], seme[b], add=True)

    def vstart(t, b):
      pltpu.async_copy(ones_v, accv.at[gi2.at[t]], semv[b], add=True)

    def ewait(t, b):
      pltpu.make_async_copy(ones_v, acce.at[si2.at[t]], seme[b]).wait()

    def vwait(t, b):
      pltpu.make_async_copy(ones_v, accv.at[gi2.at[t]], semv[b]).wait()

    for b in range(2):
      estart(b, b)
      vstart(b, b)

    def group(m, carry):
      t0 = 2 * m
      for b in range(2):
        ewait(t0 - 2 + b, b)
        vwait(t0 - 2 + b, b)
        estart(t0 + b, b)
        vstart(t0 + b, b)
      return carry

    lax.fori_loop(1, NCHD // 2, group, 0)
    for b in range(2):
      ewait(NCHD - 2 + b, b)
      vwait(NCHD - 2 + b, b)
    plsc.subcore_barrier()
    pltpu.sync_copy(acce.at[rslc], out_hbm.at[0, c, rslc])
    pltpu.sync_copy(accv.at[rslc], out_hbm.at[1, c, rslc])

  return k(gidx3, sidx3, ones_rows, zeros16)


def _sc_both_passes(table, gidx3, sidx3, zeros_half, dege):
  mesh = plsc.VectorSubcoreMesh(core_axis_name="c", subcore_axis_name="s")

  @functools.partial(
      pl.kernel,
      mesh=mesh,
      out_type=jax.ShapeDtypeStruct((2, R, DH), jnp.float32),
      compiler_params=pltpu.CompilerParams(use_tc_tiling_on_sc=False),
      scratch_types=[
          pltpu.VMEM((NCHT, CH), jnp.int32),
          pltpu.VMEM((NCHT, CH), jnp.int32),
      ] + [pltpu.VMEM((CH, DH), jnp.float32)] * NBUF
        + [pltpu.VMEM((CHT, 16), jnp.float32),
           pltpu.VMEM((CHT, 16), jnp.float32),
           pltpu.VMEM_SHARED((R, DH), jnp.float32),
           pltpu.VMEM_SHARED((R, DH), jnp.float32)]
        + [pltpu.SemaphoreType.DMA] * (2 * NBUF),
  )
  def k(table_hbm, gidx_hbm, sidx_hbm, zeros_hbm, dege_hbm, out_hbm,
        *scratch):
    gi2, si2 = scratch[0], scratch[1]
    rows = list(scratch[2:2 + NBUF])
    d0buf = scratch[2 + NBUF]
    d1buf = scratch[3 + NBUF]
    tbl = scratch[4 + NBUF]
    acc = scratch[5 + NBUF]
    gsem = list(scratch[6 + NBUF:6 + 2 * NBUF])
    ssem = list(scratch[6 + 2 * NBUF:6 + 3 * NBUF])
    c = lax.axis_index("c")
    s = lax.axis_index("s")
    rslc = pl.ds(s * RPT, RPT)
    pltpu.sync_copy(table_hbm.at[rslc, pl.ds(c * DH, DH)], tbl.at[rslc])
    pltpu.sync_copy(zeros_hbm.at[rslc], acc.at[rslc])
    pltpu.sync_copy(gidx_hbm.at[s], gi2)
    pltpu.sync_copy(sidx_hbm.at[s], si2)
    plsc.subcore_barrier()

    def run_pass(gidx, sidx):
      def gstart(t, b):
        return pltpu.async_copy(tbl.at[gidx.at[t]], rows[b], gsem[b])

      def sstart(t, b):
        return pltpu.async_copy(rows[b], acc.at[sidx.at[t]], ssem[b],
                                add=True)

      def swait(t, b):
        pltpu.make_async_copy(rows[b], acc.at[sidx.at[t]], ssem[b]).wait()

      gd = [gstart(b, b) for b in range(NBUF)]
      for b in range(NBUF):
        gd[b].wait()
        sstart(b, b)

      def group(g, carry):
        t0 = g * NBUF
        gd = []
        for b in range(NBUF):
          swait(t0 - NBUF + b, b)
          gd.append(gstart(t0 + b, b))
        for b in range(NBUF):
          gd[b].wait()
          sstart(t0 + b, b)
        return carry

      lax.fori_loop(1, NGRP, group, 0)
      for b in range(NBUF):
        swait((NGRP - 1) * NBUF + b, b)

    run_pass(gi2, si2)
    plsc.subcore_barrier()

    def transform(row0, n):
      pltpu.sync_copy(acc.at[pl.ds(row0, n)], rows[0].at[pl.ds(0, n)])
      pltpu.sync_copy(dege_hbm.at[0, pl.ds(row0, n)], d0buf.at[pl.ds(0, n)])
      pltpu.sync_copy(dege_hbm.at[1, pl.ds(row0, n)], d1buf.at[pl.ds(0, n)])

      def row_body(r, carry):
        dvec = d0buf[r, pl.ds(0, 16)] + d1buf[r, pl.ds(0, 16)]
        invvec = jnp.where(dvec > 0.0, 1.0 / dvec, 0.0)
        inv = invvec[0]
        for kcol in range(DH // 16):
          cs = pl.ds(kcol * 16, 16)
          rows[1][r, cs] = jnp.maximum(rows[0][r, cs] * inv, 0.0)
        return carry

      lax.fori_loop(0, n, row_body, 0)
      pltpu.sync_copy(rows[1].at[pl.ds(0, n)], tbl.at[pl.ds(row0, n)])

    base = s * RPT
    nfull = RPT // CHT
    for ci in range(nfull):
      transform(base + ci * CHT, CHT)
    tail = RPT - nfull * CHT
    if tail:
      transform(base + nfull * CHT, tail)
    pltpu.sync_copy(zeros_hbm.at[rslc], acc.at[rslc])
    plsc.subcore_barrier()

    run_pass(si2, gi2)
    plsc.subcore_barrier()
    pltpu.sync_copy(acc.at[rslc], out_hbm.at[c, rslc])

  return k(table, gidx3, sidx3, zeros_half, dege)


def _combine2(q0, q1, d0, d1):
  def body(q0_ref, q1_ref, d0_ref, d1_ref, o_ref):
    deg = d0_ref[...][:, :1] + d1_ref[...][:, :1]
    inv = jnp.where(deg > 0.0, 1.0 / deg, 0.0)
    y0 = jnp.maximum(q0_ref[...] * inv, 0.0)
    y1 = jnp.maximum(q1_ref[...] * inv, 0.0)
    n2 = jnp.sum(y0 * y0, axis=1, keepdims=True) + jnp.sum(
        y1 * y1, axis=1, keepdims=True)
    scale = 1.0 / jnp.maximum(jnp.sqrt(n2), 1e-12)
    o_ref[:, :DH] = y0 * scale
    o_ref[:, DH:] = y1 * scale

  grid = 25
  blk = V // grid
  return pl.pallas_call(
      body,
      grid=(grid,),
      in_specs=[pl.BlockSpec((blk, DH), lambda i: (i, 0)),
                pl.BlockSpec((blk, DH), lambda i: (i, 0)),
                pl.BlockSpec((blk, 16), lambda i: (i, 0)),
                pl.BlockSpec((blk, 16), lambda i: (i, 0))],
      out_specs=pl.BlockSpec((blk, D), lambda i: (i, 0)),
      out_shape=jax.ShapeDtypeStruct((V, D), jnp.float32),
  )(q0, q1, d0, d1)


def _tile_layout(idx):
  cap = 16 * NCHT * CH
  flat = jnp.concatenate([idx, jnp.full((cap - E,), V, jnp.int32)])
  return flat.reshape(16, NCHT, CH)


def kernel(x, edge):
  edge_j = edge[0]
  edge_i = edge[1]

  gj = _tile_layout(edge_j)
  si = _tile_layout(edge_i)

  npadd = EPADD - E
  shp = (32, NCHD, CHD)
  gd = jnp.concatenate([edge_j, jnp.full((npadd,), V, jnp.int32)]).reshape(shp)
  sd = jnp.concatenate([edge_i, jnp.full((npadd,), V, jnp.int32)]).reshape(shp)

  zeros_half = jnp.zeros((R, DH), jnp.float32)
  zeros16 = jnp.zeros((R, 16), jnp.float32)
  ones_rows = jnp.zeros((CHD, 16), jnp.float32).at[:, 0].set(1.0)
  xa = jnp.zeros((R, D), jnp.float32).at[:V].set(x)

  deg = _deg_kernel(gd, sd, ones_rows, zeros16)
  q = _sc_both_passes(xa, gj, si, zeros_half, deg[0])
  return _combine2(q[0], q[1], deg[1, 0], deg[1, 1])
